# Initial kernel scaffold; baseline (speedup 1.0000x reference)
import functools

import jax
import jax.numpy as jnp
from jax import lax
from jax.experimental import pallas as pl
from jax.experimental.pallas import tpu as pltpu
from jax.experimental.pallas import tpu_sc as plsc

N = 10000
D = 128
NC = 2
NS = 16
L = 16
C = 128
RPT = N // NS
ZB = 125


def _spmm_sc(colp, rowp, valp, h, chunks_per_worker):
    mesh = plsc.VectorSubcoreMesh(
        core_axis_name="c", subcore_axis_name="s", num_cores=NC, num_subcores=NS
    )

    @functools.partial(
        pl.kernel,
        out_type=jax.ShapeDtypeStruct((NC, N, D), jnp.float32),
        mesh=mesh,
        scratch_types=[
            pltpu.VMEM((C,), jnp.int32),
            pltpu.VMEM((C,), jnp.int32),
            pltpu.VMEM((C,), jnp.float32),
            pltpu.VMEM((C, D), jnp.float32),
            pltpu.VMEM_SHARED((N, D), jnp.float32),
            pltpu.SemaphoreType.DMA,
        ],
    )
    def spmm(col_hbm, row_hbm, val_hbm, h_hbm, part_hbm,
             colv, rowv, valv, rows_v, agg_sh, sem):
        cid = lax.axis_index("c")
        sid = lax.axis_index("s")
        wid = sid * NC + cid
        zero = jnp.zeros((L,), jnp.float32)

        def zrow(i, carry):
            for q in range(D // L):
                rows_v[i, pl.ds(q * L, L)] = zero
            return carry
        lax.fori_loop(0, C, zrow, 0)
        for k in range(RPT // ZB):
            pltpu.sync_copy(rows_v.at[pl.ds(0, ZB)],
                            agg_sh.at[pl.ds(sid * RPT + k * ZB, ZB)])
        plsc.subcore_barrier()

        def chunk(ci, carry):
            off = (wid * chunks_per_worker + ci) * C
            pltpu.sync_copy(col_hbm.at[pl.ds(off, C)], colv)
            pltpu.sync_copy(row_hbm.at[pl.ds(off, C)], rowv)
            pltpu.sync_copy(val_hbm.at[pl.ds(off, C)], valv)
            pltpu.async_copy(h_hbm.at[colv], rows_v, sem).wait()

            def scale(i, c2):
                v = valv[i]
                for q in range(D // L):
                    rows_v[i, pl.ds(q * L, L)] = rows_v[i, pl.ds(q * L, L)] * v
                return c2
            lax.fori_loop(0, C, scale, 0)

            pltpu.sync_copy(rows_v, agg_sh.at[rowv], add=True)
            return carry
        lax.fori_loop(0, chunks_per_worker, chunk, 0)
        plsc.subcore_barrier()

        for k in range(RPT // ZB):
            r0 = sid * RPT + k * ZB
            pltpu.sync_copy(agg_sh.at[pl.ds(r0, ZB)],
                            part_hbm.at[cid, pl.ds(r0, ZB), :])

    return spmm(colp, rowp, valp, h)


def _linear_relu_tc(part, W, b):
    BM = 1000

    def body(x_ref, w_ref, b_ref, o_ref):
        x = x_ref[0] + x_ref[1]
        y = lax.dot_general(x, w_ref[...], (((1,), (1,)), ((), ())),
                            preferred_element_type=jnp.float32)
        o_ref[...] = jnp.maximum(y + b_ref[...], 0.0)

    return pl.pallas_call(
        body,
        grid=(N // BM,),
        in_specs=[
            pl.BlockSpec((NC, BM, D), lambda i: (0, i, 0)),
            pl.BlockSpec((D, D), lambda i: (0, 0)),
            pl.BlockSpec((1, D), lambda i: (0, 0)),
        ],
        out_specs=pl.BlockSpec((BM, D), lambda i: (i, 0)),
        out_shape=jax.ShapeDtypeStruct((N, D), jnp.float32),
    )(part, W, b.reshape(1, D))


def kernel(A_global_edge_index, A_global_values, hidden_global, W, b):
    row = A_global_edge_index[0]
    col = A_global_edge_index[1]
    E = row.shape[0]

    per_worker = NC * NS * C
    chunks_per_worker = -(-E // per_worker)
    EP = chunks_per_worker * per_worker
    pad = EP - E
    colp = jnp.concatenate([col, jnp.zeros((pad,), col.dtype)])
    rowp = jnp.concatenate([row, jnp.zeros((pad,), row.dtype)])
    valp = jnp.concatenate([A_global_values,
                            jnp.zeros((pad,), A_global_values.dtype)])

    part = _spmm_sc(colp, rowp, valp, hidden_global, chunks_per_worker)
    return _linear_relu_tc(part, W, b)


# speedup vs baseline: 3.7609x; 3.7609x over previous
import functools

import jax
import jax.numpy as jnp
from jax import lax
from jax.experimental import pallas as pl
from jax.experimental.pallas import tpu as pltpu
from jax.experimental.pallas import tpu_sc as plsc

N = 10000
D = 128
NC = 2
NS = 16
L = 16
C = 128
NP = 10240
RPT = NP // NS
ZB = 128


def _spmm_sc(colp, rowp, valp, h, chunks_per_worker):
    mesh = plsc.VectorSubcoreMesh(
        core_axis_name="c", subcore_axis_name="s", num_cores=NC, num_subcores=NS
    )

    @functools.partial(
        pl.kernel,
        out_type=jax.ShapeDtypeStruct((NC, NP, D), jnp.float32),
        mesh=mesh,
        scratch_types=[
            pltpu.VMEM((C,), jnp.int32),
            pltpu.VMEM((C,), jnp.int32),
            pltpu.VMEM((C,), jnp.float32),
            pltpu.VMEM((C, D), jnp.float32),
            pltpu.VMEM_SHARED((NP, D), jnp.float32),
            pltpu.SemaphoreType.DMA,
        ],
    )
    def spmm(col_hbm, row_hbm, val_hbm, h_hbm, part_hbm,
             colv, rowv, valv, rows_v, agg_sh, sem):
        cid = lax.axis_index("c")
        sid = lax.axis_index("s")
        wid = sid * NC + cid
        zero = jnp.zeros((L,), jnp.float32)

        def zrow(i, carry):
            for q in range(D // L):
                rows_v[i, pl.ds(q * L, L)] = zero
            return carry
        lax.fori_loop(0, C, zrow, 0)
        for k in range(RPT // ZB):
            pltpu.sync_copy(rows_v.at[pl.ds(0, ZB)],
                            agg_sh.at[pl.ds(sid * RPT + k * ZB, ZB)])
        plsc.subcore_barrier()

        def chunk(ci, carry):
            off = (wid * chunks_per_worker + ci) * C
            pltpu.sync_copy(col_hbm.at[pl.ds(off, C)], colv)
            pltpu.sync_copy(row_hbm.at[pl.ds(off, C)], rowv)
            pltpu.sync_copy(val_hbm.at[pl.ds(off, C)], valv)
            pltpu.async_copy(h_hbm.at[colv], rows_v, sem).wait()

            def scale(g, c2):
                vv = valv[pl.ds(g * L, L)]
                for j in range(L):
                    v = vv[j]
                    i = g * L + j
                    for q in range(D // L):
                        rows_v[i, pl.ds(q * L, L)] = (
                            rows_v[i, pl.ds(q * L, L)] * v)
                return c2
            lax.fori_loop(0, C // L, scale, 0)

            pltpu.sync_copy(rows_v, agg_sh.at[rowv], add=True)
            return carry
        lax.fori_loop(0, chunks_per_worker, chunk, 0)
        plsc.subcore_barrier()

        for k in range(RPT // ZB):
            r0 = sid * RPT + k * ZB
            pltpu.sync_copy(agg_sh.at[pl.ds(r0, ZB)],
                            part_hbm.at[cid, pl.ds(r0, ZB), :])

    return spmm(colp, rowp, valp, h)


def _linear_relu_tc(part, W, b):
    BM = 1000

    def body(x_ref, w_ref, b_ref, o_ref):
        x = x_ref[0] + x_ref[1]
        y = lax.dot_general(x, w_ref[...], (((1,), (1,)), ((), ())),
                            preferred_element_type=jnp.float32)
        o_ref[...] = jnp.maximum(y + b_ref[...], 0.0)

    return pl.pallas_call(
        body,
        grid=(N // BM,),
        in_specs=[
            pl.BlockSpec((NC, BM, D), lambda i: (0, i, 0)),
            pl.BlockSpec((D, D), lambda i: (0, 0)),
            pl.BlockSpec((1, D), lambda i: (0, 0)),
        ],
        out_specs=pl.BlockSpec((BM, D), lambda i: (i, 0)),
        out_shape=jax.ShapeDtypeStruct((N, D), jnp.float32),
    )(part, W, b.reshape(1, D))


def kernel(A_global_edge_index, A_global_values, hidden_global, W, b):
    row = A_global_edge_index[0]
    col = A_global_edge_index[1]
    E = row.shape[0]

    per_worker = NC * NS * C
    chunks_per_worker = -(-E // per_worker)
    EP = chunks_per_worker * per_worker
    pad = EP - E
    colp = jnp.concatenate([col, jnp.zeros((pad,), col.dtype)])
    rowp = jnp.concatenate([row, jnp.zeros((pad,), row.dtype)])
    valp = jnp.concatenate([A_global_values,
                            jnp.zeros((pad,), A_global_values.dtype)])

    part = _spmm_sc(colp, rowp, valp, hidden_global, chunks_per_worker)
    return _linear_relu_tc(part, W, b)
